# baseline (speedup 1.0000x reference)
import jax
import jax.numpy as jnp
from jax.experimental import pallas as pl
from jax.experimental.pallas import tpu as pltpu

_B, _T, _N = 4096, 336, 21
_D, _M, _P, _K = 64, 5000, 96, 5
_BLK = 256
_HB = 256


def _split(x):
    hi = x.astype(jnp.bfloat16)
    lo = (x - hi.astype(jnp.float32)).astype(jnp.bfloat16)
    return hi, lo


def _half(x_ref, w_ref, b_ref, g_ref, be_ref, k3_ref, v2_ref,
          thr_ref, hist_ref, valid_ref, lo):
    f32 = jnp.float32
    x = x_ref[:, :, lo:lo + _HB]
    q = jnp.sum(x, axis=0) * (1.0 / _N)

    mean_val = jnp.mean(q, axis=0, keepdims=True)
    sumsq = jnp.sum(q * q, axis=0, keepdims=True)
    var = (sumsq - _T * mean_val * mean_val) * (1.0 / (_T - 1))
    std_val = jnp.maximum(jnp.sqrt(jnp.maximum(var, 0.0)), 1e-6)
    max_val = jnp.max(q, axis=0, keepdims=True)
    min_val = jnp.min(q, axis=0, keepdims=True)
    trend_val = q[_T - 1:_T, :] - q[0:1, :]

    stats = jnp.concatenate(
        [mean_val, std_val, max_val, min_val, trend_val], axis=0)
    st = jnp.swapaxes(stats, 0, 1)

    W = w_ref[...]
    h = (st[:, 0:1] * W[0:1, :] + st[:, 1:2] * W[1:2, :]
         + st[:, 2:3] * W[2:3, :] + st[:, 3:4] * W[3:4, :]
         + st[:, 4:5] * W[4:5, :] + b_ref[...])

    mu = jnp.mean(h, axis=1, keepdims=True)
    hc = h - mu
    lvar = jnp.mean(hc * hc, axis=1, keepdims=True)
    h = hc * jax.lax.rsqrt(lvar + 1e-5)
    h = h * g_ref[...] + be_ref[...]

    nrm = jnp.sqrt(jnp.sum(h * h, axis=1, keepdims=True))
    qk = h / jnp.maximum(nrm, 1e-12)

    qh, ql = _split(qk)
    k3 = k3_ref[...]
    dn = (((1,), (1,)), ((), ()))
    q3 = jnp.concatenate([qh, qh, ql, ql], axis=1)
    sim = jax.lax.dot_general(q3, k3, dn, preferred_element_type=f32)

    m0 = jnp.max(sim, axis=1, keepdims=True)
    cur = m0
    denom = jnp.ones_like(m0)
    for _ in range(_K - 1):
        sm = jnp.where(sim < cur, sim, -jnp.inf)
        cur = jnp.max(sm, axis=1, keepdims=True)
        denom = denom + jnp.exp(cur - m0)

    sel = sim >= cur
    eh = jnp.where(sel, jnp.exp(sim - m0), 0.0).astype(jnp.bfloat16)
    v2 = v2_ref[...]
    hist2 = jnp.dot(eh, v2, preferred_element_type=f32)
    hist = hist2[:, :_P] + hist2[:, _P:]

    validf = (m0 > thr_ref[lo:lo + _HB, :]).astype(f32)
    hist_ref[lo:lo + _HB, :] = hist * (validf / denom)
    valid_ref[lo:lo + _HB, :] = validf


def _body(x_ref, w_ref, b_ref, g_ref, be_ref, k3_ref, v2_ref,
          thr_ref, hist_ref, valid_ref):
    _half(x_ref, w_ref, b_ref, g_ref, be_ref, k3_ref, v2_ref,
          thr_ref, hist_ref, valid_ref, 0)


def kernel(x_normed, W, b, gamma, beta, mem_keys, mem_values, threshold_raw,
           has_extreme):
    B, T, N = x_normed.shape
    xt = jnp.transpose(x_normed, (2, 1, 0))
    thr = jnp.clip(jax.nn.sigmoid(threshold_raw)
                   - has_extreme.astype(jnp.float32) * 0.2, 0.1, None)
    thr2d = thr.reshape(B, 1)
    kh = mem_keys.astype(jnp.bfloat16)
    kl = (mem_keys - kh.astype(jnp.float32)).astype(jnp.bfloat16)
    k3 = jnp.concatenate([kh, kl, kh, kl], axis=1)
    vh = mem_values.astype(jnp.bfloat16)
    vl = (mem_values - vh.astype(jnp.float32)).astype(jnp.bfloat16)
    v2 = jnp.concatenate([vh, vl], axis=1)

    nb = B // _BLK
    hist, validf = pl.pallas_call(
        _body,
        grid=(nb,),
        in_specs=[
            pl.BlockSpec((N, T, _BLK), lambda i: (0, 0, i)),
            pl.BlockSpec((5, _D), lambda i: (0, 0)),
            pl.BlockSpec((1, _D), lambda i: (0, 0)),
            pl.BlockSpec((1, _D), lambda i: (0, 0)),
            pl.BlockSpec((1, _D), lambda i: (0, 0)),
            pl.BlockSpec((_M, 4 * _D), lambda i: (0, 0)),
            pl.BlockSpec((_M, 2 * _P), lambda i: (0, 0)),
            pl.BlockSpec((_BLK, 1), lambda i: (i, 0)),
        ],
        out_specs=[
            pl.BlockSpec((_BLK, _P), lambda i: (i, 0)),
            pl.BlockSpec((_BLK, 1), lambda i: (i, 0)),
        ],
        out_shape=[
            jax.ShapeDtypeStruct((B, _P), jnp.float32),
            jax.ShapeDtypeStruct((B, 1), jnp.float32),
        ],
        compiler_params=pltpu.CompilerParams(
            dimension_semantics=("parallel",)),
    )(xt, W, b.reshape(1, _D), gamma.reshape(1, _D), beta.reshape(1, _D),
      k3, v2, thr2d)

    return hist, validf.reshape(B) > 0.5
